# Initial kernel scaffold; baseline (speedup 1.0000x reference)
import jax
import jax.numpy as jnp
from jax.experimental import pallas as pl


def kernel(x, edge_index, W1, b1, W2, b2):
    raise NotImplementedError("write your pallas kernel here")


# speedup vs baseline: 9.5082x; 9.5082x over previous
import jax
import jax.numpy as jnp
from jax import lax
from jax.experimental import pallas as pl
from jax.experimental.pallas import tpu as pltpu
from jax.experimental.pallas import tpu_sc as plsc

N = 10000
NP = 10240
E = 320000
D_IN = 128
D_HID = 128
D_OUT = 64

NC, NS = 2, 16
NW = NC * NS
CHUNK = 80
EPW = E // NW
CPW = EPW // CHUNK
EPT = E // NS
CPT = EPT // CHUNK
RPT = NP // NS

_mesh = plsc.VectorSubcoreMesh(
    core_axis_name="c", subcore_axis_name="s", num_cores=NC, num_subcores=NS)


def _deg_body(edges_hbm, ones_hbm, zeros_hbm, out_hbm, acc, idx_v, ones_v):
    c = lax.axis_index("c")
    s = lax.axis_index("s")
    pltpu.sync_copy(zeros_hbm.at[pl.ds(s * RPT, RPT)],
                    acc.at[pl.ds(s * RPT, RPT)])
    pltpu.sync_copy(ones_hbm, ones_v)
    plsc.subcore_barrier()

    base = c * E + s * EPT

    def step(j, carry):
        start = pl.multiple_of(base + j * CHUNK, 8)
        pltpu.sync_copy(edges_hbm.at[pl.ds(start, CHUNK)], idx_v)
        pltpu.sync_copy(ones_v, acc.at[idx_v], add=True)
        return carry

    lax.fori_loop(0, CPT, step, 0)
    plsc.subcore_barrier()
    pltpu.sync_copy(acc.at[pl.ds(s * RPT, RPT)],
                    out_hbm.at[c, pl.ds(s * RPT, RPT)])


_deg_call = pl.kernel(
    _deg_body,
    out_type=jax.ShapeDtypeStruct((NC, NP, 128), jnp.float32),
    mesh=_mesh,
    scratch_types=[
        pltpu.VMEM_SHARED((NP, 128), jnp.float32),
        pltpu.VMEM((CHUNK,), jnp.int32),
        pltpu.VMEM((CHUNK, 128), jnp.float32),
    ],
)


def _scatter_body(pre_hbm, src_hbm, dst_hbm, zeros_hbm, out_hbm,
                  acc, idx_s, idx_d, rows, sem):
    c = lax.axis_index("c")
    s = lax.axis_index("s")
    w = s * NC + c
    pltpu.sync_copy(zeros_hbm.at[pl.ds(s * RPT, RPT)],
                    acc.at[pl.ds(s * RPT, RPT)])
    plsc.subcore_barrier()

    base = w * EPW

    def step(j, carry):
        start = pl.multiple_of(base + j * CHUNK, 8)
        pltpu.sync_copy(src_hbm.at[pl.ds(start, CHUNK)], idx_s)
        pltpu.sync_copy(dst_hbm.at[pl.ds(start, CHUNK)], idx_d)
        pltpu.async_copy(pre_hbm.at[idx_s], rows, sem).wait()
        pltpu.sync_copy(rows, acc.at[idx_d], add=True)
        return carry

    lax.fori_loop(0, CPW, step, 0)
    plsc.subcore_barrier()
    pltpu.sync_copy(acc.at[pl.ds(s * RPT, RPT)],
                    out_hbm.at[c, pl.ds(s * RPT, RPT)])


_scatter128 = pl.kernel(
    _scatter_body,
    out_type=jax.ShapeDtypeStruct((NC, NP, 128), jnp.float32),
    mesh=_mesh,
    scratch_types=[
        pltpu.VMEM_SHARED((NP, 128), jnp.float32),
        pltpu.VMEM((CHUNK,), jnp.int32),
        pltpu.VMEM((CHUNK,), jnp.int32),
        pltpu.VMEM((CHUNK, 128), jnp.float32),
        pltpu.SemaphoreType.DMA,
    ],
)


_BM = 1000


def _rs(deg_ref):
    return lax.rsqrt(jnp.maximum(deg_ref[0, :, 0:1], 1.0))


def _deg_spec(k):
    return pl.BlockSpec((1, _BM, 128), lambda i: (k, i, 0))


def _mm_scale_body(x_ref, w_ref, dego_ref, o_ref):
    o_ref[...] = jnp.dot(x_ref[...], w_ref[...],
                         preferred_element_type=jnp.float32) * _rs(dego_ref)


def _mm_scale(x, w, degs):
    din, dout = w.shape
    return pl.pallas_call(
        _mm_scale_body,
        grid=(N // _BM,),
        in_specs=[
            pl.BlockSpec((_BM, din), lambda i: (i, 0)),
            pl.BlockSpec((din, dout), lambda i: (0, 0)),
            _deg_spec(0),
        ],
        out_specs=pl.BlockSpec((_BM, dout), lambda i: (i, 0)),
        out_shape=jax.ShapeDtypeStruct((N, dout), jnp.float32),
    )(x, w, degs)


def _combine_mm_body(p_ref, degi_ref, b_ref, w_ref, dego_ref, o_ref):
    h = (p_ref[0] + p_ref[1]) * _rs(degi_ref) + b_ref[...]
    h = jnp.maximum(h, 0.0)
    o_ref[...] = jnp.dot(h, w_ref[...],
                         preferred_element_type=jnp.float32) * _rs(dego_ref)


def _combine_mm(parts, degs, b, w):
    din, dout = w.shape
    return pl.pallas_call(
        _combine_mm_body,
        grid=(N // _BM,),
        in_specs=[
            pl.BlockSpec((NC, _BM, din), lambda i: (0, i, 0)),
            _deg_spec(1),
            pl.BlockSpec((1, din), lambda i: (0, 0)),
            pl.BlockSpec((din, dout), lambda i: (0, 0)),
            _deg_spec(0),
        ],
        out_specs=pl.BlockSpec((_BM, dout), lambda i: (i, 0)),
        out_shape=jax.ShapeDtypeStruct((NP, dout), jnp.float32),
    )(parts, degs, b, w, degs)


def _final_body(p_ref, degi_ref, b_ref, o_ref):
    v = (p_ref[0] + p_ref[1])[:, :D_OUT]
    o_ref[...] = v * _rs(degi_ref) + b_ref[...]


def _final(parts, degs, b):
    return pl.pallas_call(
        _final_body,
        grid=(N // _BM,),
        in_specs=[
            pl.BlockSpec((NC, _BM, D_HID), lambda i: (0, i, 0)),
            _deg_spec(1),
            pl.BlockSpec((1, D_OUT), lambda i: (0, 0)),
        ],
        out_specs=pl.BlockSpec((_BM, D_OUT), lambda i: (i, 0)),
        out_shape=jax.ShapeDtypeStruct((N, D_OUT), jnp.float32),
    )(parts, degs, b)


@jax.jit
def kernel(x, edge_index, W1, b1, W2, b2):
    src = edge_index[0]
    dst = edge_index[1]
    edges_flat = edge_index.reshape(-1)

    ones128 = jnp.ones((CHUNK, 128), jnp.float32)
    zeros128 = jnp.zeros((NP, 128), jnp.float32)
    degs = _deg_call(edges_flat, ones128, zeros128)

    pre1 = _mm_scale(x, W1, degs)
    part1 = _scatter128(pre1, src, dst, zeros128)
    W2p = jnp.pad(W2, ((0, 0), (0, D_HID - D_OUT)))
    pre2 = _combine_mm(part1, degs, b1.reshape(1, D_HID), W2p)
    part2 = _scatter128(pre2, src, dst, zeros128)
    return _final(part2, degs, b2.reshape(1, D_OUT))


# speedup vs baseline: 16.3227x; 1.7167x over previous
import jax
import jax.numpy as jnp
from jax import lax
from jax.experimental import pallas as pl
from jax.experimental.pallas import tpu as pltpu
from jax.experimental.pallas import tpu_sc as plsc

N = 10000
NP = 10240
E = 320000
D_IN = 128
D_HID = 128
D_OUT = 64

NC, NS = 2, 16
NW = NC * NS
CHUNK = 80
EPW = E // NW
CPW = EPW // CHUNK
EPT = E // NS
CPT = EPT // CHUNK
RPT = NP // NS
NBUF = 4
GRP = CPW // NBUF
TAIL = CPW % NBUF
GRPD = CPT // NBUF
TAILD = CPT % NBUF

_mesh = plsc.VectorSubcoreMesh(
    core_axis_name="c", subcore_axis_name="s", num_cores=NC, num_subcores=NS)


def _deg_body(edges_hbm, ones_hbm, zeros_hbm, out_hbm,
              acc, idx_v, ones_v, sem_i, sem_s):
    c = lax.axis_index("c")
    s = lax.axis_index("s")
    pltpu.sync_copy(zeros_hbm.at[pl.ds(s * RPT, RPT)],
                    acc.at[pl.ds(s * RPT, RPT)])
    pltpu.sync_copy(ones_hbm, ones_v)
    plsc.subcore_barrier()

    base = c * E + s * EPT

    def do_group(chunk0, nb):
        di = [pltpu.async_copy(
                  edges_hbm.at[pl.ds(
                      pl.multiple_of(base + (chunk0 + b) * CHUNK, 8), CHUNK)],
                  idx_v.at[b], sem_i)
              for b in range(nb)]
        for d in di:
            d.wait()
        ds_ = [pltpu.async_copy(ones_v, acc.at[idx_v.at[b]], sem_s, add=True)
               for b in range(nb)]
        for d in ds_:
            d.wait()

    def group(g, carry):
        do_group(g * NBUF, NBUF)
        return carry

    lax.fori_loop(0, GRPD, group, 0)
    do_group(GRPD * NBUF, TAILD)
    plsc.subcore_barrier()
    pltpu.sync_copy(acc.at[pl.ds(s * RPT, RPT)],
                    out_hbm.at[c, pl.ds(s * RPT, RPT)])


_deg_call = pl.kernel(
    _deg_body,
    out_type=jax.ShapeDtypeStruct((NC, NP, 128), jnp.float32),
    mesh=_mesh,
    scratch_types=[
        pltpu.VMEM_SHARED((NP, 128), jnp.float32),
        pltpu.VMEM((NBUF, CHUNK), jnp.int32),
        pltpu.VMEM((CHUNK, 128), jnp.float32),
        pltpu.SemaphoreType.DMA,
        pltpu.SemaphoreType.DMA,
    ],
)


def _scatter_body(pre_hbm, src_hbm, dst_hbm, zeros_hbm, out_hbm,
                  acc, idx_s, idx_d, rows, sem_i, sem_g, sem_s):
    c = lax.axis_index("c")
    s = lax.axis_index("s")
    w = s * NC + c
    pltpu.sync_copy(zeros_hbm.at[pl.ds(s * RPT, RPT)],
                    acc.at[pl.ds(s * RPT, RPT)])
    plsc.subcore_barrier()

    base = w * EPW

    def do_group(chunk0, nb):
        di = []
        for b in range(nb):
            st = pl.multiple_of(base + (chunk0 + b) * CHUNK, 8)
            di.append(pltpu.async_copy(src_hbm.at[pl.ds(st, CHUNK)],
                                       idx_s.at[b], sem_i))
            di.append(pltpu.async_copy(dst_hbm.at[pl.ds(st, CHUNK)],
                                       idx_d.at[b], sem_i))
        for d in di:
            d.wait()
        dg = [pltpu.async_copy(pre_hbm.at[idx_s.at[b]], rows.at[b], sem_g)
              for b in range(nb)]
        for d in dg:
            d.wait()
        ds_ = [pltpu.async_copy(rows.at[b], acc.at[idx_d.at[b]], sem_s,
                                add=True)
               for b in range(nb)]
        for d in ds_:
            d.wait()

    def group(g, carry):
        do_group(g * NBUF, NBUF)
        return carry

    lax.fori_loop(0, GRP, group, 0)
    do_group(GRP * NBUF, TAIL)
    plsc.subcore_barrier()
    pltpu.sync_copy(acc.at[pl.ds(s * RPT, RPT)],
                    out_hbm.at[c, pl.ds(s * RPT, RPT)])


_scatter128 = pl.kernel(
    _scatter_body,
    out_type=jax.ShapeDtypeStruct((NC, NP, 128), jnp.float32),
    mesh=_mesh,
    scratch_types=[
        pltpu.VMEM_SHARED((NP, 128), jnp.float32),
        pltpu.VMEM((NBUF, CHUNK), jnp.int32),
        pltpu.VMEM((NBUF, CHUNK), jnp.int32),
        pltpu.VMEM((NBUF, CHUNK, 128), jnp.float32),
        pltpu.SemaphoreType.DMA,
        pltpu.SemaphoreType.DMA,
        pltpu.SemaphoreType.DMA,
    ],
)


_BM = 1000


def _rs(deg_ref):
    return lax.rsqrt(jnp.maximum(deg_ref[0, :, 0:1], 1.0))


def _deg_spec(k):
    return pl.BlockSpec((1, _BM, 128), lambda i: (k, i, 0))


def _mm_scale_body(x_ref, w_ref, dego_ref, o_ref):
    o_ref[...] = jnp.dot(x_ref[...], w_ref[...],
                         preferred_element_type=jnp.float32) * _rs(dego_ref)


def _mm_scale(x, w, degs):
    din, dout = w.shape
    return pl.pallas_call(
        _mm_scale_body,
        grid=(N // _BM,),
        in_specs=[
            pl.BlockSpec((_BM, din), lambda i: (i, 0)),
            pl.BlockSpec((din, dout), lambda i: (0, 0)),
            _deg_spec(0),
        ],
        out_specs=pl.BlockSpec((_BM, dout), lambda i: (i, 0)),
        out_shape=jax.ShapeDtypeStruct((N, dout), jnp.float32),
    )(x, w, degs)


def _combine_mm_body(p_ref, degi_ref, b_ref, w_ref, dego_ref, o_ref):
    h = (p_ref[0] + p_ref[1]) * _rs(degi_ref) + b_ref[...]
    h = jnp.maximum(h, 0.0)
    o_ref[...] = jnp.dot(h, w_ref[...],
                         preferred_element_type=jnp.float32) * _rs(dego_ref)


def _combine_mm(parts, degs, b, w):
    din, dout = w.shape
    return pl.pallas_call(
        _combine_mm_body,
        grid=(N // _BM,),
        in_specs=[
            pl.BlockSpec((NC, _BM, din), lambda i: (0, i, 0)),
            _deg_spec(1),
            pl.BlockSpec((1, din), lambda i: (0, 0)),
            pl.BlockSpec((din, dout), lambda i: (0, 0)),
            _deg_spec(0),
        ],
        out_specs=pl.BlockSpec((_BM, dout), lambda i: (i, 0)),
        out_shape=jax.ShapeDtypeStruct((NP, dout), jnp.float32),
    )(parts, degs, b, w, degs)


def _final_body(p_ref, degi_ref, b_ref, o_ref):
    v = (p_ref[0] + p_ref[1])[:, :D_OUT]
    o_ref[...] = v * _rs(degi_ref) + b_ref[...]


def _final(parts, degs, b):
    return pl.pallas_call(
        _final_body,
        grid=(N // _BM,),
        in_specs=[
            pl.BlockSpec((NC, _BM, D_HID), lambda i: (0, i, 0)),
            _deg_spec(1),
            pl.BlockSpec((1, D_OUT), lambda i: (0, 0)),
        ],
        out_specs=pl.BlockSpec((_BM, D_OUT), lambda i: (i, 0)),
        out_shape=jax.ShapeDtypeStruct((N, D_OUT), jnp.float32),
    )(parts, degs, b)


@jax.jit
def kernel(x, edge_index, W1, b1, W2, b2):
    src = edge_index[0]
    dst = edge_index[1]
    edges_flat = edge_index.reshape(-1)

    ones128 = jnp.ones((CHUNK, 128), jnp.float32)
    zeros128 = jnp.zeros((NP, 128), jnp.float32)
    degs = _deg_call(edges_flat, ones128, zeros128)

    pre1 = _mm_scale(x, W1, degs)
    part1 = _scatter128(pre1, src, dst, zeros128)
    W2p = jnp.pad(W2, ((0, 0), (0, D_HID - D_OUT)))
    pre2 = _combine_mm(part1, degs, b1.reshape(1, D_HID), W2p)
    part2 = _scatter128(pre2, src, dst, zeros128)
    return _final(part2, degs, b2.reshape(1, D_OUT))


# speedup vs baseline: 18.2009x; 1.1151x over previous
import jax
import jax.numpy as jnp
from jax import lax
from jax.experimental import pallas as pl
from jax.experimental.pallas import tpu as pltpu
from jax.experimental.pallas import tpu_sc as plsc

N = 10000
NP = 10240
E = 320000
D_IN = 128
D_HID = 128
D_OUT = 64

NC, NS = 2, 16
NW = NC * NS
CHUNK = 80
EPW = E // NW
CPW = EPW // CHUNK
EPT = E // NS
CPT = EPT // CHUNK
RPT = NP // NS
NBUF = 4
GRP = CPW // NBUF
TAIL = CPW % NBUF
GRPD = CPT // NBUF
TAILD = CPT % NBUF

_mesh = plsc.VectorSubcoreMesh(
    core_axis_name="c", subcore_axis_name="s", num_cores=NC, num_subcores=NS)


def _deg_body(edges_hbm, ones_hbm, zeros_hbm, out_hbm,
              acc, idx_v, ones_v, semi0, semi1, sems):
    c = lax.axis_index("c")
    s = lax.axis_index("s")
    pltpu.sync_copy(zeros_hbm.at[pl.ds(s * RPT, RPT)],
                    acc.at[pl.ds(s * RPT, RPT)])
    pltpu.sync_copy(ones_hbm, ones_v)
    plsc.subcore_barrier()

    base = c * E + s * EPT
    semi = (semi0, semi1)

    def fire_idx(g, p):
        for b in range(NBUF):
            raw = base + (g * NBUF + b) * CHUNK
            st = pl.multiple_of(jnp.minimum(raw, 2 * E - CHUNK), 8)
            pltpu.async_copy(edges_hbm.at[pl.ds(st, CHUNK)],
                             idx_v.at[p, b], semi[p])

    def work(g, p, next_g):
        if next_g is not None:
            fire_idx(next_g, 1 - p)
        for b in range(NBUF):
            pltpu.make_async_copy(edges_hbm.at[pl.ds(0, CHUNK)],
                                  idx_v.at[p, b], semi[p]).wait()
        ds_ = [pltpu.async_copy(ones_v, acc.at[idx_v.at[p, b]], sems,
                                add=True)
               for b in range(NBUF)]
        for d in ds_:
            d.wait()

    fire_idx(0, 0)

    def pair(m, carry):
        work(2 * m, 0, 2 * m + 1)
        work(2 * m + 1, 1, 2 * m + 2)
        return carry

    lax.fori_loop(0, GRPD // 2 - 1, pair, 0)
    work(GRPD - 2, 0, GRPD - 1)
    work(GRPD - 1, 1, None)
    for t in range(TAILD):
        st = pl.multiple_of(base + (GRPD * NBUF + t) * CHUNK, 8)
        pltpu.sync_copy(edges_hbm.at[pl.ds(st, CHUNK)], idx_v.at[0, 0])
        pltpu.sync_copy(ones_v, acc.at[idx_v.at[0, 0]], add=True)

    plsc.subcore_barrier()
    pltpu.sync_copy(acc.at[pl.ds(s * RPT, RPT)],
                    out_hbm.at[c, pl.ds(s * RPT, RPT)])


_deg_call = pl.kernel(
    _deg_body,
    out_type=jax.ShapeDtypeStruct((NC, NP, 128), jnp.float32),
    mesh=_mesh,
    scratch_types=[
        pltpu.VMEM_SHARED((NP, 128), jnp.float32),
        pltpu.VMEM((2, NBUF, CHUNK), jnp.int32),
        pltpu.VMEM((CHUNK, 128), jnp.float32),
        pltpu.SemaphoreType.DMA,
        pltpu.SemaphoreType.DMA,
        pltpu.SemaphoreType.DMA,
    ],
)


def _scatter_body(pre_hbm, src_hbm, dst_hbm, zeros_hbm, out_hbm,
                  acc, idx_s, idx_d, rows,
                  semi0, semi1, semg0, semg1, semg2, semg3, sems):
    c = lax.axis_index("c")
    s = lax.axis_index("s")
    w = s * NC + c
    pltpu.sync_copy(zeros_hbm.at[pl.ds(s * RPT, RPT)],
                    acc.at[pl.ds(s * RPT, RPT)])
    plsc.subcore_barrier()

    base = w * EPW
    semi = (semi0, semi1)
    semg = (semg0, semg1, semg2, semg3)

    def fire_idx(g, p):
        for b in range(NBUF):
            raw = base + (g * NBUF + b) * CHUNK
            st = pl.multiple_of(jnp.minimum(raw, E - CHUNK), 8)
            pltpu.async_copy(src_hbm.at[pl.ds(st, CHUNK)],
                             idx_s.at[p, b], semi[p])
            pltpu.async_copy(dst_hbm.at[pl.ds(st, CHUNK)],
                             idx_d.at[p, b], semi[p])

    def work(g, p, next_g):
        if next_g is not None:
            fire_idx(next_g, 1 - p)
        for b in range(NBUF):
            pltpu.make_async_copy(src_hbm.at[pl.ds(0, CHUNK)],
                                  idx_s.at[p, b], semi[p]).wait()
            pltpu.make_async_copy(dst_hbm.at[pl.ds(0, CHUNK)],
                                  idx_d.at[p, b], semi[p]).wait()
        dg = [pltpu.async_copy(pre_hbm.at[idx_s.at[p, b]], rows.at[b],
                               semg[b])
              for b in range(NBUF)]
        ds_ = []
        for b in range(NBUF):
            dg[b].wait()
            ds_.append(pltpu.async_copy(rows.at[b], acc.at[idx_d.at[p, b]],
                                        sems, add=True))
        for d in ds_:
            d.wait()

    fire_idx(0, 0)

    def pair(m, carry):
        work(2 * m, 0, 2 * m + 1)
        work(2 * m + 1, 1, 2 * m + 2)
        return carry

    lax.fori_loop(0, GRP // 2, pair, 0)
    work(GRP - 1, 0, None)
    for t in range(TAIL):
        st = pl.multiple_of(base + (GRP * NBUF + t) * CHUNK, 8)
        pltpu.sync_copy(src_hbm.at[pl.ds(st, CHUNK)], idx_s.at[0, 0])
        pltpu.sync_copy(dst_hbm.at[pl.ds(st, CHUNK)], idx_d.at[0, 0])
        pltpu.async_copy(pre_hbm.at[idx_s.at[0, 0]], rows.at[0],
                         semg[0]).wait()
        pltpu.sync_copy(rows.at[0], acc.at[idx_d.at[0, 0]], add=True)

    plsc.subcore_barrier()
    pltpu.sync_copy(acc.at[pl.ds(s * RPT, RPT)],
                    out_hbm.at[c, pl.ds(s * RPT, RPT)])


_scatter128 = pl.kernel(
    _scatter_body,
    out_type=jax.ShapeDtypeStruct((NC, NP, 128), jnp.float32),
    mesh=_mesh,
    scratch_types=[
        pltpu.VMEM_SHARED((NP, 128), jnp.float32),
        pltpu.VMEM((2, NBUF, CHUNK), jnp.int32),
        pltpu.VMEM((2, NBUF, CHUNK), jnp.int32),
        pltpu.VMEM((NBUF, CHUNK, 128), jnp.float32),
        pltpu.SemaphoreType.DMA,
        pltpu.SemaphoreType.DMA,
        pltpu.SemaphoreType.DMA,
        pltpu.SemaphoreType.DMA,
        pltpu.SemaphoreType.DMA,
        pltpu.SemaphoreType.DMA,
        pltpu.SemaphoreType.DMA,
    ],
)


_BM = 1000


def _rs(deg_ref):
    return lax.rsqrt(jnp.maximum(deg_ref[0, :, 0:1], 1.0))


def _deg_spec(k):
    return pl.BlockSpec((1, _BM, 128), lambda i: (k, i, 0))


def _mm_scale_body(x_ref, w_ref, dego_ref, o_ref):
    o_ref[...] = jnp.dot(x_ref[...], w_ref[...],
                         preferred_element_type=jnp.float32) * _rs(dego_ref)


def _mm_scale(x, w, degs):
    din, dout = w.shape
    return pl.pallas_call(
        _mm_scale_body,
        grid=(N // _BM,),
        in_specs=[
            pl.BlockSpec((_BM, din), lambda i: (i, 0)),
            pl.BlockSpec((din, dout), lambda i: (0, 0)),
            _deg_spec(0),
        ],
        out_specs=pl.BlockSpec((_BM, dout), lambda i: (i, 0)),
        out_shape=jax.ShapeDtypeStruct((N, dout), jnp.float32),
    )(x, w, degs)


def _combine_mm_body(p_ref, degi_ref, b_ref, w_ref, dego_ref, o_ref):
    h = (p_ref[0] + p_ref[1]) * _rs(degi_ref) + b_ref[...]
    h = jnp.maximum(h, 0.0)
    o_ref[...] = jnp.dot(h, w_ref[...],
                         preferred_element_type=jnp.float32) * _rs(dego_ref)


def _combine_mm(parts, degs, b, w):
    din, dout = w.shape
    return pl.pallas_call(
        _combine_mm_body,
        grid=(N // _BM,),
        in_specs=[
            pl.BlockSpec((NC, _BM, din), lambda i: (0, i, 0)),
            _deg_spec(1),
            pl.BlockSpec((1, din), lambda i: (0, 0)),
            pl.BlockSpec((din, dout), lambda i: (0, 0)),
            _deg_spec(0),
        ],
        out_specs=pl.BlockSpec((_BM, dout), lambda i: (i, 0)),
        out_shape=jax.ShapeDtypeStruct((NP, dout), jnp.float32),
    )(parts, degs, b, w, degs)


def _final_body(p_ref, degi_ref, b_ref, o_ref):
    v = (p_ref[0] + p_ref[1])[:, :D_OUT]
    o_ref[...] = v * _rs(degi_ref) + b_ref[...]


def _final(parts, degs, b):
    return pl.pallas_call(
        _final_body,
        grid=(N // _BM,),
        in_specs=[
            pl.BlockSpec((NC, _BM, D_HID), lambda i: (0, i, 0)),
            _deg_spec(1),
            pl.BlockSpec((1, D_OUT), lambda i: (0, 0)),
        ],
        out_specs=pl.BlockSpec((_BM, D_OUT), lambda i: (i, 0)),
        out_shape=jax.ShapeDtypeStruct((N, D_OUT), jnp.float32),
    )(parts, degs, b)


@jax.jit
def kernel(x, edge_index, W1, b1, W2, b2):
    src = edge_index[0]
    dst = edge_index[1]
    edges_flat = edge_index.reshape(-1)

    ones128 = jnp.ones((CHUNK, 128), jnp.float32)
    zeros128 = jnp.zeros((NP, 128), jnp.float32)
    degs = _deg_call(edges_flat, ones128, zeros128)

    pre1 = _mm_scale(x, W1, degs)
    part1 = _scatter128(pre1, src, dst, zeros128)
    W2p = jnp.pad(W2, ((0, 0), (0, D_HID - D_OUT)))
    pre2 = _combine_mm(part1, degs, b1.reshape(1, D_HID), W2p)
    part2 = _scatter128(pre2, src, dst, zeros128)
    return _final(part2, degs, b2.reshape(1, D_OUT))


# speedup vs baseline: 18.2111x; 1.0006x over previous
import jax
import jax.numpy as jnp
from jax import lax
from jax.experimental import pallas as pl
from jax.experimental.pallas import tpu as pltpu
from jax.experimental.pallas import tpu_sc as plsc

N = 10000
NP = 10240
E = 320000
D_IN = 128
D_HID = 128
D_OUT = 64

NC, NS = 2, 16
NW = NC * NS
CHUNK = 80
EPW = E // NW
CPW = EPW // CHUNK
EPT = E // NS
CPT = EPT // CHUNK
RPT = NP // NS
NBUF = 4
GRP = CPW // NBUF
TAIL = CPW % NBUF
GRPD = CPT // NBUF
TAILD = CPT % NBUF

_mesh = plsc.VectorSubcoreMesh(
    core_axis_name="c", subcore_axis_name="s", num_cores=NC, num_subcores=NS)


def _deg_body(edges_hbm, ones_hbm, zeros_hbm, out_hbm,
              acc, idx_v, ones_v, semi0, semi1, sems):
    c = lax.axis_index("c")
    s = lax.axis_index("s")
    pltpu.sync_copy(zeros_hbm.at[pl.ds(s * RPT, RPT)],
                    acc.at[pl.ds(s * RPT, RPT)])
    pltpu.sync_copy(ones_hbm, ones_v)
    plsc.subcore_barrier()

    base = c * E + s * EPT
    semi = (semi0, semi1)

    def fire_idx(g, p):
        for b in range(NBUF):
            raw = base + (g * NBUF + b) * CHUNK
            st = pl.multiple_of(jnp.minimum(raw, 2 * E - CHUNK), 8)
            pltpu.async_copy(edges_hbm.at[pl.ds(st, CHUNK)],
                             idx_v.at[p, b], semi[p])

    def drain_scatters(p):
        for b in range(NBUF):
            pltpu.make_async_copy(ones_v, acc.at[idx_v.at[p, b]],
                                  sems).wait()

    def work(g, p, next_g, drain_prev):
        for b in range(NBUF):
            pltpu.make_async_copy(edges_hbm.at[pl.ds(0, CHUNK)],
                                  idx_v.at[p, b], semi[p]).wait()
        if drain_prev:
            drain_scatters(p)
        if next_g is not None:
            fire_idx(next_g, 1 - p)
        for b in range(NBUF):
            pltpu.async_copy(ones_v, acc.at[idx_v.at[p, b]], sems,
                             add=True)

    fire_idx(0, 0)
    work(0, 0, 1, False)

    def pair(m, carry):
        work(2 * m + 1, 1, 2 * m + 2, True)
        work(2 * m + 2, 0, 2 * m + 3, True)
        return carry

    lax.fori_loop(0, GRPD // 2 - 1, pair, 0)
    work(GRPD - 1, 1, None, True)
    drain_scatters(1)
    for t in range(TAILD):
        st = pl.multiple_of(base + (GRPD * NBUF + t) * CHUNK, 8)
        pltpu.sync_copy(edges_hbm.at[pl.ds(st, CHUNK)], idx_v.at[0, 0])
        pltpu.sync_copy(ones_v, acc.at[idx_v.at[0, 0]], add=True)

    plsc.subcore_barrier()
    pltpu.sync_copy(acc.at[pl.ds(s * RPT, RPT)],
                    out_hbm.at[c, pl.ds(s * RPT, RPT)])


_deg_call = pl.kernel(
    _deg_body,
    out_type=jax.ShapeDtypeStruct((NC, NP, 128), jnp.float32),
    mesh=_mesh,
    scratch_types=[
        pltpu.VMEM_SHARED((NP, 128), jnp.float32),
        pltpu.VMEM((2, NBUF, CHUNK), jnp.int32),
        pltpu.VMEM((CHUNK, 128), jnp.float32),
        pltpu.SemaphoreType.DMA,
        pltpu.SemaphoreType.DMA,
        pltpu.SemaphoreType.DMA,
    ],
)


def _scatter_body(pre_hbm, src_hbm, dst_hbm, zeros_hbm, out_hbm,
                  acc, idx_s, idx_d, rows,
                  semi0, semi1, semg0, semg1, semg2, semg3, sems):
    c = lax.axis_index("c")
    s = lax.axis_index("s")
    w = s * NC + c
    pltpu.sync_copy(zeros_hbm.at[pl.ds(s * RPT, RPT)],
                    acc.at[pl.ds(s * RPT, RPT)])
    plsc.subcore_barrier()

    base = w * EPW
    semi = (semi0, semi1)
    semg = (semg0, semg1, semg2, semg3)

    def fire_idx(g, p):
        for b in range(NBUF):
            raw = base + (g * NBUF + b) * CHUNK
            st = pl.multiple_of(jnp.minimum(raw, E - CHUNK), 8)
            pltpu.async_copy(src_hbm.at[pl.ds(st, CHUNK)],
                             idx_s.at[p, b], semi[p])
            pltpu.async_copy(dst_hbm.at[pl.ds(st, CHUNK)],
                             idx_d.at[p, b], semi[p])

    def drain_scatters(p):
        for b in range(NBUF):
            pltpu.make_async_copy(rows.at[b], acc.at[idx_d.at[p, b]],
                                  sems).wait()

    def work(g, p, next_g, drain_prev):
        for b in range(NBUF):
            pltpu.make_async_copy(src_hbm.at[pl.ds(0, CHUNK)],
                                  idx_s.at[p, b], semi[p]).wait()
            pltpu.make_async_copy(dst_hbm.at[pl.ds(0, CHUNK)],
                                  idx_d.at[p, b], semi[p]).wait()
        if drain_prev:
            drain_scatters(p)
        if next_g is not None:
            fire_idx(next_g, 1 - p)
        dg = [pltpu.async_copy(pre_hbm.at[idx_s.at[p, b]], rows.at[b],
                               semg[b])
              for b in range(NBUF)]
        for b in range(NBUF):
            dg[b].wait()
            pltpu.async_copy(rows.at[b], acc.at[idx_d.at[p, b]],
                             sems, add=True)

    fire_idx(0, 0)
    work(0, 0, 1, False)

    def pair(m, carry):
        work(2 * m + 1, 1, 2 * m + 2, True)
        work(2 * m + 2, 0, 2 * m + 3, True)
        return carry

    lax.fori_loop(0, GRP // 2 - 1, pair, 0)
    work(GRP - 2, 1, GRP - 1, True)
    work(GRP - 1, 0, None, True)
    drain_scatters(0)
    for t in range(TAIL):
        st = pl.multiple_of(base + (GRP * NBUF + t) * CHUNK, 8)
        pltpu.sync_copy(src_hbm.at[pl.ds(st, CHUNK)], idx_s.at[0, 0])
        pltpu.sync_copy(dst_hbm.at[pl.ds(st, CHUNK)], idx_d.at[0, 0])
        pltpu.async_copy(pre_hbm.at[idx_s.at[0, 0]], rows.at[0],
                         semg[0]).wait()
        pltpu.sync_copy(rows.at[0], acc.at[idx_d.at[0, 0]], add=True)

    plsc.subcore_barrier()
    pltpu.sync_copy(acc.at[pl.ds(s * RPT, RPT)],
                    out_hbm.at[c, pl.ds(s * RPT, RPT)])


_scatter128 = pl.kernel(
    _scatter_body,
    out_type=jax.ShapeDtypeStruct((NC, NP, 128), jnp.float32),
    mesh=_mesh,
    scratch_types=[
        pltpu.VMEM_SHARED((NP, 128), jnp.float32),
        pltpu.VMEM((2, NBUF, CHUNK), jnp.int32),
        pltpu.VMEM((2, NBUF, CHUNK), jnp.int32),
        pltpu.VMEM((NBUF, CHUNK, 128), jnp.float32),
        pltpu.SemaphoreType.DMA,
        pltpu.SemaphoreType.DMA,
        pltpu.SemaphoreType.DMA,
        pltpu.SemaphoreType.DMA,
        pltpu.SemaphoreType.DMA,
        pltpu.SemaphoreType.DMA,
        pltpu.SemaphoreType.DMA,
    ],
)


_BM = 1000


def _rs(deg_ref):
    return lax.rsqrt(jnp.maximum(deg_ref[0, :, 0:1], 1.0))


def _deg_spec(k):
    return pl.BlockSpec((1, _BM, 128), lambda i: (k, i, 0))


def _mm_scale_body(x_ref, w_ref, dego_ref, o_ref):
    o_ref[...] = jnp.dot(x_ref[...], w_ref[...],
                         preferred_element_type=jnp.float32) * _rs(dego_ref)


def _mm_scale(x, w, degs):
    din, dout = w.shape
    return pl.pallas_call(
        _mm_scale_body,
        grid=(N // _BM,),
        in_specs=[
            pl.BlockSpec((_BM, din), lambda i: (i, 0)),
            pl.BlockSpec((din, dout), lambda i: (0, 0)),
            _deg_spec(0),
        ],
        out_specs=pl.BlockSpec((_BM, dout), lambda i: (i, 0)),
        out_shape=jax.ShapeDtypeStruct((N, dout), jnp.float32),
    )(x, w, degs)


def _combine_mm_body(p_ref, degi_ref, b_ref, w_ref, dego_ref, o_ref):
    h = (p_ref[0] + p_ref[1]) * _rs(degi_ref) + b_ref[...]
    h = jnp.maximum(h, 0.0)
    o_ref[...] = jnp.dot(h, w_ref[...],
                         preferred_element_type=jnp.float32) * _rs(dego_ref)


def _combine_mm(parts, degs, b, w):
    din, dout = w.shape
    return pl.pallas_call(
        _combine_mm_body,
        grid=(N // _BM,),
        in_specs=[
            pl.BlockSpec((NC, _BM, din), lambda i: (0, i, 0)),
            _deg_spec(1),
            pl.BlockSpec((1, din), lambda i: (0, 0)),
            pl.BlockSpec((din, dout), lambda i: (0, 0)),
            _deg_spec(0),
        ],
        out_specs=pl.BlockSpec((_BM, dout), lambda i: (i, 0)),
        out_shape=jax.ShapeDtypeStruct((NP, dout), jnp.float32),
    )(parts, degs, b, w, degs)


def _final_body(p_ref, degi_ref, b_ref, o_ref):
    v = (p_ref[0] + p_ref[1])[:, :D_OUT]
    o_ref[...] = v * _rs(degi_ref) + b_ref[...]


def _final(parts, degs, b):
    return pl.pallas_call(
        _final_body,
        grid=(N // _BM,),
        in_specs=[
            pl.BlockSpec((NC, _BM, D_HID), lambda i: (0, i, 0)),
            _deg_spec(1),
            pl.BlockSpec((1, D_OUT), lambda i: (0, 0)),
        ],
        out_specs=pl.BlockSpec((_BM, D_OUT), lambda i: (i, 0)),
        out_shape=jax.ShapeDtypeStruct((N, D_OUT), jnp.float32),
    )(parts, degs, b)


@jax.jit
def kernel(x, edge_index, W1, b1, W2, b2):
    src = edge_index[0]
    dst = edge_index[1]
    edges_flat = edge_index.reshape(-1)

    ones128 = jnp.ones((CHUNK, 128), jnp.float32)
    zeros128 = jnp.zeros((NP, 128), jnp.float32)
    degs = _deg_call(edges_flat, ones128, zeros128)

    pre1 = _mm_scale(x, W1, degs)
    part1 = _scatter128(pre1, src, dst, zeros128)
    W2p = jnp.pad(W2, ((0, 0), (0, D_HID - D_OUT)))
    pre2 = _combine_mm(part1, degs, b1.reshape(1, D_HID), W2p)
    part2 = _scatter128(pre2, src, dst, zeros128)
    return _final(part2, degs, b2.reshape(1, D_OUT))
